# Initial kernel scaffold; baseline (speedup 1.0000x reference)
import jax
import jax.numpy as jnp
from jax.experimental import pallas as pl


def kernel(adj, features, W_neigh, W_lin):
    raise NotImplementedError("write your pallas kernel here")


# speedup vs baseline: 1.9977x; 1.9977x over previous
import jax
import jax.numpy as jnp
from jax.experimental import pallas as pl
from jax.experimental.pallas import tpu as pltpu

_BM = 400


def _sage_kernel(adj_ref, feat_ref, wn_ref, wl_ref, out_ref):
    i = pl.program_id(0)
    d = feat_ref.shape[1]

    t = jnp.dot(adj_ref[...], feat_ref[...], preferred_element_type=jnp.float32)

    rs = jnp.sum(adj_ref[...], axis=1, keepdims=True) + 1.0

    wl1 = wl_ref[:, :d]
    wl2 = wl_ref[:, d:]
    k = jnp.dot(wl2, wn_ref[...], preferred_element_type=jnp.float32)
    f_blk = feat_ref[pl.ds(i * _BM, _BM), :]
    dn = (((1,), (1,)), ((), ()))
    self_term = jax.lax.dot_general(f_blk, wl1, dn, preferred_element_type=jnp.float32)
    neigh_term = jax.lax.dot_general(t / rs, k, dn, preferred_element_type=jnp.float32)
    out_ref[...] = self_term + neigh_term


def kernel(adj, features, W_neigh, W_lin):
    n, d = features.shape
    d_out = W_lin.shape[0]
    grid = (n // _BM,)
    return pl.pallas_call(
        _sage_kernel,
        grid=grid,
        in_specs=[
            pl.BlockSpec((_BM, n), lambda i: (i, 0)),
            pl.BlockSpec((n, d), lambda i: (0, 0)),
            pl.BlockSpec((d, d), lambda i: (0, 0)),
            pl.BlockSpec((d_out, 2 * d), lambda i: (0, 0)),
        ],
        out_specs=pl.BlockSpec((_BM, d_out), lambda i: (i, 0)),
        out_shape=jax.ShapeDtypeStruct((n, d_out), jnp.float32),
        compiler_params=pltpu.CompilerParams(
            dimension_semantics=("arbitrary",),
        ),
    )(adj, features, W_neigh, W_lin)
